# baseline (speedup 1.0000x reference)
import functools

import jax
import jax.numpy as jnp
from jax import lax
from jax.experimental import pallas as pl
from jax.experimental.pallas import tpu as pltpu
from jax.experimental.pallas import tpu_sc as plsc

B = 16384
F = 128
L = 16
NC = 2
NS = 16
NW = NC * NS
BPW = B // NW
CHUNK = 64
NCHUNK = BPW // CHUNK


def _make_sc_kernel():
    mesh = plsc.VectorSubcoreMesh(core_axis_name="c", subcore_axis_name="s")

    @functools.partial(
        pl.kernel,
        out_type=jax.ShapeDtypeStruct((B,), jnp.float32),
        mesh=mesh,
        compiler_params=pltpu.CompilerParams(needs_layout_passes=False),
        scratch_types=[
            pltpu.VMEM((BPW,), jnp.int32),
            pltpu.VMEM((BPW,), jnp.int32),
            pltpu.VMEM((2, CHUNK, F), jnp.float32),
            pltpu.VMEM((2, CHUNK, F), jnp.float32),
            pltpu.VMEM((BPW + 8,), jnp.float32),
            pltpu.SemaphoreType.DMA,
            pltpu.SemaphoreType.DMA,
        ],
    )
    def sc_kernel(users_hbm, item_hbm, uemb_hbm, iemb_hbm, out_hbm,
                  uidx, iidx, urows, irows, outv, usem, isem):
        wid = lax.axis_index("s") * NC + lax.axis_index("c")
        base = wid * BPW

        cu = pltpu.async_copy(users_hbm.at[pl.ds(base, BPW)], uidx, usem)
        ci = pltpu.async_copy(item_hbm.at[pl.ds(base, BPW)], iidx, isem)
        cu.wait()
        ci.wait()

        def start(c):
            buf = c % 2
            return (
                pltpu.async_copy(uemb_hbm.at[uidx.at[pl.ds(c * CHUNK, CHUNK)]],
                                 urows.at[buf], usem),
                pltpu.async_copy(iemb_hbm.at[iidx.at[pl.ds(c * CHUNK, CHUNK)]],
                                 irows.at[buf], isem),
            )

        pending = start(0)
        for c in range(NCHUNK):
            cur = pending
            if c + 1 < NCHUNK:
                pending = start(c + 1)
            cur[0].wait()
            cur[1].wait()
            buf = c % 2
            ub = urows.at[buf]
            ib = irows.at[buf]
            lane = lax.iota(jnp.int32, L)

            def gbody(g, c=c, ub=ub, ib=ib, lane=lane):
                parts = [
                    ub[g, pl.ds(k * L, L)] * ib[g, pl.ds(k * L, L)]
                    for k in range(F // L)
                ]
                while len(parts) > 1:
                    parts = [a + b for a, b in zip(parts[::2], parts[1::2])]
                total = plsc.cumsum(parts[0])
                plsc.store_compressed(outv.at[pl.ds(c * CHUNK + g, L)],
                                      total, mask=lane == L - 1)

            plsc.parallel_loop(0, CHUNK, 1, unroll=2)(gbody)

        pltpu.sync_copy(outv.at[pl.ds(0, BPW)], out_hbm.at[pl.ds(base, BPW)])

    return sc_kernel


_sc_kernel = _make_sc_kernel()


def kernel(users, item, user_emb, item_emb):
    return _sc_kernel(users, item, user_emb, item_emb)


# speedup vs baseline: 1.0786x; 1.0786x over previous
import functools

import jax
import jax.numpy as jnp
from jax import lax
from jax.experimental import pallas as pl
from jax.experimental.pallas import tpu as pltpu
from jax.experimental.pallas import tpu_sc as plsc

B = 16384
F = 128
L = 16
NC = 2
NS = 16
NW = NC * NS
BPW = B // NW
CHUNK = 128
NCHUNK = BPW // CHUNK


def _make_sc_kernel():
    mesh = plsc.VectorSubcoreMesh(core_axis_name="c", subcore_axis_name="s")

    @functools.partial(
        pl.kernel,
        out_type=jax.ShapeDtypeStruct((B,), jnp.float32),
        mesh=mesh,
        compiler_params=pltpu.CompilerParams(needs_layout_passes=False),
        scratch_types=[
            pltpu.VMEM((BPW,), jnp.int32),
            pltpu.VMEM((BPW,), jnp.int32),
            pltpu.VMEM((3, CHUNK, F), jnp.float32),
            pltpu.VMEM((3, CHUNK, F), jnp.float32),
            pltpu.VMEM((BPW + 8,), jnp.float32),
            pltpu.SemaphoreType.DMA,
            pltpu.SemaphoreType.DMA,
        ],
    )
    def sc_kernel(users_hbm, item_hbm, uemb_hbm, iemb_hbm, out_hbm,
                  uidx, iidx, urows, irows, outv, usem, isem):
        wid = lax.axis_index("s") * NC + lax.axis_index("c")
        base = wid * BPW

        cu = pltpu.async_copy(users_hbm.at[pl.ds(base, BPW)], uidx, usem)
        ci = pltpu.async_copy(item_hbm.at[pl.ds(base, BPW)], iidx, isem)
        cu.wait()
        ci.wait()

        def start(c):
            buf = c % 3
            return (
                pltpu.async_copy(uemb_hbm.at[uidx.at[pl.ds(c * CHUNK, CHUNK)]],
                                 urows.at[buf], usem),
                pltpu.async_copy(iemb_hbm.at[iidx.at[pl.ds(c * CHUNK, CHUNK)]],
                                 irows.at[buf], isem),
            )

        inflight = [start(0), start(1)]
        for c in range(NCHUNK):
            cur = inflight.pop(0)
            if c + 2 < NCHUNK:
                inflight.append(start(c + 2))
            cur[0].wait()
            cur[1].wait()
            buf = c % 3
            ub = urows.at[buf]
            ib = irows.at[buf]
            lane = lax.iota(jnp.int32, L)

            def gbody(g, c=c, ub=ub, ib=ib, lane=lane):
                parts = [
                    ub[g, pl.ds(k * L, L)] * ib[g, pl.ds(k * L, L)]
                    for k in range(F // L)
                ]
                while len(parts) > 1:
                    parts = [a + b for a, b in zip(parts[::2], parts[1::2])]
                total = plsc.cumsum(parts[0])
                plsc.store_compressed(outv.at[pl.ds(c * CHUNK + g, L)],
                                      total, mask=lane == L - 1)

            plsc.parallel_loop(0, CHUNK, 1, unroll=2)(gbody)

        pltpu.sync_copy(outv.at[pl.ds(0, BPW)], out_hbm.at[pl.ds(base, BPW)])

    return sc_kernel


_sc_kernel = _make_sc_kernel()


def kernel(users, item, user_emb, item_emb):
    return _sc_kernel(users, item, user_emb, item_emb)
